# baseline (speedup 1.0000x reference)
import functools

import jax
import jax.numpy as jnp
from jax import lax
from jax.experimental import pallas as pl
from jax.experimental.pallas import tpu as pltpu
from jax.experimental.pallas import tpu_sc as plsc

_B = 16
_C = 64
_HW = 1024
_K = 1024
_BETA = 0.25

_NC = 2
_NS = 16
_NW = _NC * _NS
_N = _B * _HW
_PPW = _N // _NW
_LANE = 16
_GCH = 128
_NCHUNK = _PPW // _GCH


def _argmin_body(z_ref, w_ref, idx_ref, loss_ref):
    b = pl.program_id(0)
    zb = z_ref[0]
    w = w_ref[...]
    s = jax.lax.dot_general(w, zb, (((1,), (0,)), ((), ())),
                            preferred_element_type=jnp.float32)
    w2 = jnp.sum(w * w, axis=1, keepdims=True)
    z2 = jnp.sum(zb * zb, axis=0, keepdims=True)
    d = (z2 + w2) - 2.0 * s
    m = jnp.min(d, axis=0, keepdims=True)
    ii = jax.lax.broadcasted_iota(jnp.int32, (_K, _HW), 0)
    idx = jnp.min(jnp.where(d == m, ii, _K), axis=0).astype(jnp.int32)
    idx_ref[0, 0, :] = idx
    part = jnp.sum(m, axis=1, keepdims=True)

    @pl.when(b == 0)
    def _():
        loss_ref[...] = jnp.zeros((1, 1), jnp.float32)

    loss_ref[...] += part


@jax.jit
def _vq_argmin_tc(z3, W):
    return pl.pallas_call(
        _argmin_body,
        grid=(_B,),
        in_specs=[
            pl.BlockSpec((1, _C, _HW), lambda b: (b, 0, 0)),
            pl.BlockSpec((_K, _C), lambda b: (0, 0)),
        ],
        out_specs=[
            pl.BlockSpec((1, 1, _HW), lambda b: (b, 0, 0)),
            pl.BlockSpec((1, 1), lambda b: (0, 0)),
        ],
        out_shape=[
            jax.ShapeDtypeStruct((_B, 1, _HW), jnp.int32),
            jax.ShapeDtypeStruct((1, 1), jnp.float32),
        ],
    )(z3, W)


def _sc_gather_body(w_hbm, idx_hbm, out_hbm, idx_v, rows_v, t_v, sem):
    wid = lax.axis_index("s") * _NC + lax.axis_index("c")
    pltpu.sync_copy(idx_hbm.at[pl.ds(wid * _NCHUNK, _NCHUNK), :], idx_v)
    copies = [
        pltpu.async_copy(w_hbm.at[idx_v.at[k]],
                         rows_v.at[pl.ds(k * _GCH, _GCH), :], sem)
        for k in range(_NCHUNK)
    ]
    for c in copies:
        c.wait()
    lane512 = lax.iota(jnp.int32, _LANE) * _PPW

    def body(r, carry):
        for g in range(_C // _LANE):
            vec = rows_v[r, pl.ds(g * _LANE, _LANE)]
            plsc.store_scatter(t_v, [lane512 + (g * _LANE * _PPW + r)], vec)
        return carry

    lax.fori_loop(0, _PPW, body, 0)
    b = wid // (_HW // _PPW)
    p0 = (wid % (_HW // _PPW)) * _PPW
    outs = [
        pltpu.async_copy(t_v.at[pl.ds(c * _PPW, _PPW)],
                         out_hbm.at[b, c, pl.ds(p0, _PPW)], sem)
        for c in range(_C)
    ]
    for c in outs:
        c.wait()


@jax.jit
def _vq_gather_sc(W, idx2):
    f = functools.partial(
        pl.kernel,
        mesh=plsc.VectorSubcoreMesh(core_axis_name="c", subcore_axis_name="s"),
        compiler_params=pltpu.CompilerParams(needs_layout_passes=False),
        out_type=jax.ShapeDtypeStruct((_B, _C, _HW), jnp.float32),
        scratch_types=[
            pltpu.VMEM((_NCHUNK, _GCH), jnp.int32),
            pltpu.VMEM((_PPW, 2 * _C), jnp.float32),
            pltpu.VMEM((_C * _PPW,), jnp.float32),
            pltpu.SemaphoreType.DMA,
        ],
    )(_sc_gather_body)
    return f(W, idx2)


def kernel(z, W):
    z3 = z.reshape(_B, _C, _HW)
    idx3, loss = _vq_argmin_tc(z3, W)
    w_pad = jnp.pad(W, ((0, 0), (0, _C)))
    zq3 = _vq_gather_sc(w_pad, idx3.reshape(_N // _GCH, _GCH))
    vq_loss = loss[0, 0] * ((1.0 + _BETA) / (_B * _C * _HW))
    return zq3.reshape(z.shape), vq_loss, idx3.reshape(_N)
